# Initial kernel scaffold; baseline (speedup 1.0000x reference)
import jax
import jax.numpy as jnp
from jax.experimental import pallas as pl


def kernel(token_hidden, expert_idx, slot_idx, expert_offsets):
    raise NotImplementedError("write your pallas kernel here")


# speedup vs baseline: 1.3944x; 1.3944x over previous
import functools

import jax
import jax.numpy as jnp
from jax import lax
from jax.experimental import pallas as pl
from jax.experimental.pallas import tpu as pltpu
from jax.experimental.pallas import tpu_sc as plsc

_NUM_TOKENS = 8192
_HIDDEN = 2048
_NUM_EXPERTS = 16
_CAPACITY = 1024
_ROWS = _NUM_EXPERTS * _CAPACITY

_L = 16
_NW = 32
_RPW = _ROWS // _NW
_CH = 16
_NCHUNK = _RPW // _CH


def _body(tok_hbm, e_hbm, s_hbm, off_hbm, out_hbm,
          e_v, s_v, off_v, inv_v, idx_v, gbuf, zbuf, sem):
    cid = lax.axis_index("c")
    sid = lax.axis_index("s")
    wid = sid * 2 + cid
    r0 = wid * _RPW

    pltpu.sync_copy(e_hbm, e_v)
    pltpu.sync_copy(s_hbm, s_v)
    pltpu.sync_copy(off_hbm, off_v)

    zi = jnp.zeros((_L,), jnp.int32)
    zf = jnp.zeros((_L,), jnp.float32)

    @pl.loop(0, _RPW // _L)
    def _zero_inv(i):
        inv_v[pl.ds(i * _L, _L)] = zi

    for r in range(_CH):
        @pl.loop(0, _HIDDEN // _L)
        def _zero_zbuf(j):
            zbuf[r, pl.ds(j * _L, _L)] = zf

    iota = lax.iota(jnp.int32, _L)
    off_all = off_v[pl.ds(0, _L)]

    def p1(t, cnt):
        e = e_v[pl.ds(t * _L, _L)]
        s = s_v[pl.ds(t * _L, _L)]
        off = jnp.take_along_axis(off_all, e, axis=0, mode="promise_in_bounds")
        rel = off + s - r0
        m = (rel >= 0) & (rel < _RPW)
        relc = jnp.where(m, rel, 0)
        vals = t * _L + iota + 1
        plsc.store_scatter(inv_v, [relc], vals, mask=m)
        return cnt + jnp.where(m, 1, 0)

    cnt_vec = lax.fori_loop(0, _NUM_TOKENS // _L, p1, jnp.zeros((_L,), jnp.int32))
    nvalid = jnp.sum(cnt_vec)

    @pl.loop(0, _RPW // _L)
    def _mk_idx(i):
        v = inv_v[pl.ds(i * _L, _L)]
        idx_v[pl.ds(i * _L, _L)] = jnp.maximum(v - 1, 0)

    nfull = nvalid // _CH
    rem = nvalid - nfull * _CH

    @pl.loop(0, _NCHUNK)
    def _chunk(c):
        dst = out_hbm.at[pl.ds(r0 + c * _CH, _CH)]

        @pl.when(c < nfull)
        def _full():
            pltpu.async_copy(tok_hbm.at[idx_v.at[pl.ds(c * _CH, _CH)]],
                             gbuf, sem).wait()
            pltpu.sync_copy(gbuf, dst)

        @pl.when((c == nfull) & (rem > 0))
        def _partial():
            pltpu.async_copy(tok_hbm.at[idx_v.at[pl.ds(c * _CH, _CH)]],
                             gbuf, sem).wait()
            for r in range(_CH):
                @pl.when(r >= rem)
                def _zrow():
                    @pl.loop(0, _HIDDEN // _L)
                    def _z(j):
                        gbuf[r, pl.ds(j * _L, _L)] = zf
            pltpu.sync_copy(gbuf, dst)

        @pl.when((c > nfull) | ((c == nfull) & (rem == 0)))
        def _empty():
            pltpu.sync_copy(zbuf, dst)


@functools.partial(jax.jit, static_argnames=())
def _dispatch(token_hidden, expert_idx, slot_idx, expert_offsets16):
    mesh = plsc.VectorSubcoreMesh(core_axis_name="c", subcore_axis_name="s")
    return pl.kernel(
        _body,
        out_type=jax.ShapeDtypeStruct((_ROWS, _HIDDEN), jnp.float32),
        mesh=mesh,
        compiler_params=pltpu.CompilerParams(needs_layout_passes=False),
        scratch_types=[
            pltpu.VMEM((_NUM_TOKENS,), jnp.int32),
            pltpu.VMEM((_NUM_TOKENS,), jnp.int32),
            pltpu.VMEM((_NUM_EXPERTS,), jnp.int32),
            pltpu.VMEM((_RPW,), jnp.int32),
            pltpu.VMEM((_RPW,), jnp.int32),
            pltpu.VMEM((_CH, _HIDDEN), jnp.float32),
            pltpu.VMEM((_CH, _HIDDEN), jnp.float32),
            pltpu.SemaphoreType.DMA,
        ],
    )(token_hidden, expert_idx, slot_idx, expert_offsets16)


def kernel(token_hidden, expert_idx, slot_idx, expert_offsets):
    off16 = expert_offsets[:_NUM_EXPERTS].astype(jnp.int32)
    return _dispatch(token_hidden,
                     expert_idx.astype(jnp.int32),
                     slot_idx.astype(jnp.int32),
                     off16)
